# Initial kernel scaffold; baseline (speedup 1.0000x reference)
import jax
import jax.numpy as jnp
from jax.experimental import pallas as pl


def kernel(x, edge_attr, edge_index, W_edge, b_edge, W_conv, ln_w, ln_b):
    raise NotImplementedError("write your pallas kernel here")


# speedup vs baseline: 1.6024x; 1.6024x over previous
import numpy as np
import jax
import jax.numpy as jnp
from jax.experimental import pallas as pl
from jax.experimental.pallas import tpu as pltpu

_N = 384
_C = 64
_H = 60
_L = 4
_E = 2304
_ECH = 5
_F = _C * _H

_EB = 576

_PREC = jax.lax.Precision.HIGHEST


def _transform_kernel(hs_ref, ea_ref, wconv_ref, wedge_ref, bedge_ref,
                      A_ref, B_ref, c_ref):
    wconv = wconv_ref[...]
    W1 = wconv[:, :_C]
    W2 = wconv[:, _C:2 * _C]
    W3 = wconv[:, 2 * _C:]
    hs = hs_ref[...]
    A_ref[...] = jax.lax.dot(W1 - W2, hs, precision=_PREC,
                             preferred_element_type=jnp.float32)
    B_ref[...] = jax.lax.dot(W2, hs, precision=_PREC,
                             preferred_element_type=jnp.float32)
    t = jax.lax.dot_general(ea_ref[...], wedge_ref[...],
                            (((1,), (1,)), ((), ())), precision=_PREC,
                            preferred_element_type=jnp.float32)
    t = t + bedge_ref[...]
    c_ref[...] = jax.lax.dot_general(t, W3, (((1,), (1,)), ((), ())),
                                     precision=_PREC,
                                     preferred_element_type=jnp.float32)


def _edge_kernel(A_ref, B_ref, c_ref, dst_ref, src_ref, rep_ref,
                 lnw_ref, lnb_ref, out_ref):
    dst = dst_ref[...]
    src = src_ref[...]
    iota_n = jax.lax.broadcasted_iota(jnp.int32, (_EB, _N), 1)
    Pd = (iota_n == dst).astype(jnp.float32)
    Ps = (iota_n == src).astype(jnp.float32)
    mA = jax.lax.dot(Pd, A_ref[...], precision=_PREC,
                     preferred_element_type=jnp.float32)
    mB = jax.lax.dot(Ps, B_ref[...], precision=_PREC,
                     preferred_element_type=jnp.float32)
    cb = jax.lax.dot(c_ref[...], rep_ref[...], precision=_PREC,
                     preferred_element_type=jnp.float32)
    m = mA + mB + cb
    mu = jnp.mean(m, axis=1, keepdims=True)
    d = m - mu
    var = jnp.mean(d * d, axis=1, keepdims=True)
    mn = d * jax.lax.rsqrt(var + 1e-5)
    mn = mn * lnw_ref[...] + lnb_ref[...]
    mact = jnp.maximum(mn, 0.0)
    contrib = jax.lax.dot_general(Pd, mact, (((0,), (0,)), ((), ())),
                                  precision=_PREC,
                                  preferred_element_type=jnp.float32)

    @pl.when(pl.program_id(0) == 0)
    def _init():
        out_ref[...] = jnp.zeros_like(out_ref)

    out_ref[...] += contrib


_transform_call = pl.pallas_call(
    _transform_kernel,
    out_shape=(
        jax.ShapeDtypeStruct((_C, _N * _H), jnp.float32),
        jax.ShapeDtypeStruct((_C, _N * _H), jnp.float32),
        jax.ShapeDtypeStruct((_E, _C), jnp.float32),
    ),
)

_edge_call = pl.pallas_call(
    _edge_kernel,
    grid=(_E // _EB,),
    in_specs=[
        pl.BlockSpec((_N, _F), lambda i: (0, 0)),
        pl.BlockSpec((_N, _F), lambda i: (0, 0)),
        pl.BlockSpec((_EB, _C), lambda i: (i, 0)),
        pl.BlockSpec((_EB, 1), lambda i: (i, 0)),
        pl.BlockSpec((_EB, 1), lambda i: (i, 0)),
        pl.BlockSpec((_C, _F), lambda i: (0, 0)),
        pl.BlockSpec((1, _F), lambda i: (0, 0)),
        pl.BlockSpec((1, _F), lambda i: (0, 0)),
    ],
    out_specs=pl.BlockSpec((_N, _F), lambda i: (0, 0)),
    out_shape=jax.ShapeDtypeStruct((_N, _F), jnp.float32),
)

_REP = jnp.asarray(np.repeat(np.eye(_C, dtype=np.float32), _H, axis=1))


def kernel(x, edge_attr, edge_index, W_edge, b_edge, W_conv, ln_w, ln_b):
    src = edge_index[0].reshape(_E, 1)
    dst = edge_index[1].reshape(_E, 1)
    hs = x.reshape(_N, _C, _H).transpose(1, 0, 2).reshape(_C, _N * _H)
    out = None
    for l in range(_L):
        A, B, c = _transform_call(hs, edge_attr, W_conv[l], W_edge[l],
                                  b_edge[l].reshape(1, _C))
        A_nm = A.reshape(_C, _N, _H).transpose(1, 0, 2).reshape(_N, _F)
        B_nm = B.reshape(_C, _N, _H).transpose(1, 0, 2).reshape(_N, _F)
        out = _edge_call(A_nm, B_nm, c, dst, src, _REP,
                         ln_w[l].reshape(1, _F), ln_b[l].reshape(1, _F))
        if l < _L - 1:
            hs = out.reshape(_N, _C, _H).transpose(1, 0, 2).reshape(_C, _N * _H)
    return out


# speedup vs baseline: 2.1312x; 1.3300x over previous
import numpy as np
import jax
import jax.numpy as jnp
from jax.experimental import pallas as pl
from jax.experimental.pallas import tpu as pltpu

_N = 384
_C = 64
_H = 60
_L = 4
_E = 2304
_ECH = 5
_F = _C * _H
_K = 2 * _N + _C

_EB = 576

_HIGHEST = jax.lax.Precision.HIGHEST


def _split16(v):
    hi = v.astype(jnp.bfloat16)
    lo = (v - hi.astype(jnp.float32)).astype(jnp.bfloat16)
    return hi, lo


def _transform_kernel(hs_ref, ea_ref, wconv_ref, wedge_ref, bedge_ref,
                      Ahi_ref, Alo_ref, Bhi_ref, Blo_ref, chi_ref, clo_ref):
    wconv = wconv_ref[...]
    W1 = wconv[:, :_C]
    W2 = wconv[:, _C:2 * _C]
    W3 = wconv[:, 2 * _C:]
    hs = hs_ref[...]
    A = jax.lax.dot(W1 - W2, hs, precision=_HIGHEST,
                    preferred_element_type=jnp.float32)
    B = jax.lax.dot(W2, hs, precision=_HIGHEST,
                    preferred_element_type=jnp.float32)
    Ahi_ref[...], Alo_ref[...] = _split16(A)
    Bhi_ref[...], Blo_ref[...] = _split16(B)
    t = jax.lax.dot_general(ea_ref[...], wedge_ref[...],
                            (((1,), (1,)), ((), ())), precision=_HIGHEST,
                            preferred_element_type=jnp.float32)
    t = t + bedge_ref[...]
    c = jax.lax.dot_general(t, W3, (((1,), (1,)), ((), ())),
                            precision=_HIGHEST,
                            preferred_element_type=jnp.float32)
    chi_ref[...], clo_ref[...] = _split16(c)


def _edge_kernel(VAhi_ref, VAlo_ref, chi_ref, clo_ref, dst_ref, src_ref,
                 lnw_ref, lnb_ref, out_ref):
    dst = dst_ref[...]
    src = src_ref[...]
    iota_n = jax.lax.broadcasted_iota(jnp.int32, (_EB, _N), 1)
    Pd = (iota_n == dst).astype(jnp.bfloat16)
    Ps = (iota_n == src).astype(jnp.bfloat16)
    lhs_hi = jnp.concatenate([Pd, Ps, chi_ref[...]], axis=1)
    lhs_lo = jnp.concatenate([Pd, Ps, clo_ref[...]], axis=1)
    m = (jax.lax.dot(lhs_hi, VAhi_ref[...],
                     preferred_element_type=jnp.float32)
         + jax.lax.dot(lhs_lo, VAlo_ref[...],
                       preferred_element_type=jnp.float32))
    mu = jnp.mean(m, axis=1, keepdims=True)
    d = m - mu
    var = jnp.mean(d * d, axis=1, keepdims=True)
    mn = d * jax.lax.rsqrt(var + 1e-5)
    mn = mn * lnw_ref[...] + lnb_ref[...]
    mact = jnp.maximum(mn, 0.0)
    ahi, alo = _split16(mact)
    contrib = (jax.lax.dot_general(Pd, ahi, (((0,), (0,)), ((), ())),
                                   preferred_element_type=jnp.float32)
               + jax.lax.dot_general(Pd, alo, (((0,), (0,)), ((), ())),
                                     preferred_element_type=jnp.float32))

    @pl.when(pl.program_id(0) == 0)
    def _init():
        out_ref[...] = jnp.zeros_like(out_ref)

    out_ref[...] += contrib


_transform_call = pl.pallas_call(
    _transform_kernel,
    out_shape=(
        jax.ShapeDtypeStruct((_C, _N * _H), jnp.bfloat16),
        jax.ShapeDtypeStruct((_C, _N * _H), jnp.bfloat16),
        jax.ShapeDtypeStruct((_C, _N * _H), jnp.bfloat16),
        jax.ShapeDtypeStruct((_C, _N * _H), jnp.bfloat16),
        jax.ShapeDtypeStruct((_E, _C), jnp.bfloat16),
        jax.ShapeDtypeStruct((_E, _C), jnp.bfloat16),
    ),
)

_edge_call = pl.pallas_call(
    _edge_kernel,
    grid=(_E // _EB,),
    in_specs=[
        pl.BlockSpec((_K, _F), lambda i: (0, 0)),
        pl.BlockSpec((_K, _F), lambda i: (0, 0)),
        pl.BlockSpec((_EB, _C), lambda i: (i, 0)),
        pl.BlockSpec((_EB, _C), lambda i: (i, 0)),
        pl.BlockSpec((_EB, 1), lambda i: (i, 0)),
        pl.BlockSpec((_EB, 1), lambda i: (i, 0)),
        pl.BlockSpec((1, _F), lambda i: (0, 0)),
        pl.BlockSpec((1, _F), lambda i: (0, 0)),
    ],
    out_specs=pl.BlockSpec((_N, _F), lambda i: (0, 0)),
    out_shape=jax.ShapeDtypeStruct((_N, _F), jnp.float32),
)

_REP = jnp.asarray(np.repeat(np.eye(_C, dtype=np.float32), _H, axis=1),
                   dtype=jnp.bfloat16)


def _to_node_major(a_cm):
    return a_cm.reshape(_C, _N, _H).transpose(1, 0, 2).reshape(_N, _F)


def kernel(x, edge_attr, edge_index, W_edge, b_edge, W_conv, ln_w, ln_b):
    src = edge_index[0].reshape(_E, 1)
    dst = edge_index[1].reshape(_E, 1)
    hs = x.reshape(_N, _C, _H).transpose(1, 0, 2).reshape(_C, _N * _H)
    out = None
    for l in range(_L):
        Ahi, Alo, Bhi, Blo, chi, clo = _transform_call(
            hs, edge_attr, W_conv[l], W_edge[l], b_edge[l].reshape(1, _C))
        VAhi = jnp.concatenate(
            [_to_node_major(Ahi), _to_node_major(Bhi), _REP], axis=0)
        VAlo = jnp.concatenate(
            [_to_node_major(Alo), _to_node_major(Blo), _REP], axis=0)
        out = _edge_call(VAhi, VAlo, chi, clo, dst, src,
                         ln_w[l].reshape(1, _F), ln_b[l].reshape(1, _F))
        if l < _L - 1:
            hs = out.reshape(_N, _C, _H).transpose(1, 0, 2).reshape(_C, _N * _H)
    return out


# speedup vs baseline: 3.9655x; 1.8607x over previous
import numpy as np
import jax
import jax.numpy as jnp
from jax.experimental import pallas as pl
from jax.experimental.pallas import tpu as pltpu

_N = 384
_C = 64
_H = 60
_L = 4
_E = 2304
_ECH = 5
_F = _C * _H
_NF = _N * _H
_K = 2 * _N + _C

_EB = 576

_HIGHEST = jax.lax.Precision.HIGHEST


def _split16(v):
    hi = v.astype(jnp.bfloat16)
    lo = (v - hi.astype(jnp.float32)).astype(jnp.bfloat16)
    return hi, lo


_RB = 2880


def _transform_kernel(hf_ref, wconv_ref, Ahi_ref, Alo_ref, Bhi_ref, Blo_ref):
    wconv = wconv_ref[...]
    W1 = wconv[:, :_C]
    W2 = wconv[:, _C:2 * _C]
    Wab = jnp.concatenate([W1 - W2, W2], axis=0)
    AB = jax.lax.dot_general(hf_ref[...], Wab, (((1,), (1,)), ((), ())),
                             precision=_HIGHEST,
                             preferred_element_type=jnp.float32)
    hi, lo = _split16(AB)
    Ahi_ref[...] = hi[:, :_C]
    Alo_ref[...] = lo[:, :_C]
    Bhi_ref[...] = hi[:, _C:]
    Blo_ref[...] = lo[:, _C:]


def _edge_kernel(VAhi_ref, VAlo_ref, ea_ref, wedge_ref, bedge_ref, w3_ref,
                 dst_ref, src_ref, lnw_ref, lnb_ref, out_ref):
    dst = dst_ref[...]
    src = src_ref[...]
    iota_n = jax.lax.broadcasted_iota(jnp.int32, (_EB, _N), 1)
    Pd = (iota_n == dst).astype(jnp.bfloat16)
    Ps = (iota_n == src).astype(jnp.bfloat16)
    t = jax.lax.dot_general(ea_ref[...], wedge_ref[...],
                            (((1,), (1,)), ((), ())), precision=_HIGHEST,
                            preferred_element_type=jnp.float32)
    t = t + bedge_ref[...]
    c = jax.lax.dot_general(t, w3_ref[...], (((1,), (1,)), ((), ())),
                            precision=_HIGHEST,
                            preferred_element_type=jnp.float32)
    chi, clo = _split16(c)
    lhs_hi = jnp.concatenate([Pd, Ps, chi], axis=1)
    lhs_lo = jnp.concatenate([Pd, Ps, clo], axis=1)
    m = (jax.lax.dot(lhs_hi, VAhi_ref[...],
                     preferred_element_type=jnp.float32)
         + jax.lax.dot(lhs_lo, VAlo_ref[...],
                       preferred_element_type=jnp.float32))
    mu = jnp.mean(m, axis=1, keepdims=True)
    d = m - mu
    var = jnp.mean(d * d, axis=1, keepdims=True)
    mn = d * jax.lax.rsqrt(var + 1e-5)
    mn = mn * lnw_ref[...] + lnb_ref[...]
    mact = jnp.maximum(mn, 0.0)
    ahi, alo = _split16(mact)
    contrib = (jax.lax.dot_general(Pd, ahi, (((0,), (0,)), ((), ())),
                                   preferred_element_type=jnp.float32)
               + jax.lax.dot_general(Pd, alo, (((0,), (0,)), ((), ())),
                                     preferred_element_type=jnp.float32))

    @pl.when(pl.program_id(0) == 0)
    def _init():
        out_ref[...] = jnp.zeros_like(out_ref)

    out_ref[...] += contrib


_transform_call = pl.pallas_call(
    _transform_kernel,
    grid=(_NF // _RB,),
    in_specs=[
        pl.BlockSpec((_RB, _C), lambda i: (i, 0)),
        pl.BlockSpec((_C, 3 * _C), lambda i: (0, 0)),
    ],
    out_specs=(
        pl.BlockSpec((_RB, _C), lambda i: (i, 0)),
        pl.BlockSpec((_RB, _C), lambda i: (i, 0)),
        pl.BlockSpec((_RB, _C), lambda i: (i, 0)),
        pl.BlockSpec((_RB, _C), lambda i: (i, 0)),
    ),
    out_shape=(
        jax.ShapeDtypeStruct((_NF, _C), jnp.bfloat16),
        jax.ShapeDtypeStruct((_NF, _C), jnp.bfloat16),
        jax.ShapeDtypeStruct((_NF, _C), jnp.bfloat16),
        jax.ShapeDtypeStruct((_NF, _C), jnp.bfloat16),
    ),
)

_edge_call = pl.pallas_call(
    _edge_kernel,
    grid=(_E // _EB,),
    in_specs=[
        pl.BlockSpec((_K, _F), lambda i: (0, 0)),
        pl.BlockSpec((_K, _F), lambda i: (0, 0)),
        pl.BlockSpec((_EB, _ECH), lambda i: (i, 0)),
        pl.BlockSpec((_C, _ECH), lambda i: (0, 0)),
        pl.BlockSpec((1, _C), lambda i: (0, 0)),
        pl.BlockSpec((_C, _C), lambda i: (0, 0)),
        pl.BlockSpec((_EB, 1), lambda i: (i, 0)),
        pl.BlockSpec((_EB, 1), lambda i: (i, 0)),
        pl.BlockSpec((1, _F), lambda i: (0, 0)),
        pl.BlockSpec((1, _F), lambda i: (0, 0)),
    ],
    out_specs=pl.BlockSpec((_N, _F), lambda i: (0, 0)),
    out_shape=jax.ShapeDtypeStruct((_N, _F), jnp.float32),
)

_REP_NP = np.tile(np.eye(_C, dtype=np.float32), (1, _H))


def kernel(x, edge_attr, edge_index, W_edge, b_edge, W_conv, ln_w, ln_b):
    src = edge_index[0].reshape(_E, 1)
    dst = edge_index[1].reshape(_E, 1)
    rep = jnp.asarray(_REP_NP, dtype=jnp.bfloat16)
    h = x.reshape(_N, _C, _H).transpose(0, 2, 1).reshape(_N, _F)
    lnw = jnp.transpose(ln_w, (0, 2, 1)).reshape(_L, 1, _F)
    lnb = jnp.transpose(ln_b, (0, 2, 1)).reshape(_L, 1, _F)
    for l in range(_L):
        Ahi, Alo, Bhi, Blo = _transform_call(h.reshape(_NF, _C), W_conv[l])
        VAhi = jnp.concatenate(
            [Ahi.reshape(_N, _F), Bhi.reshape(_N, _F), rep], axis=0)
        VAlo = jnp.concatenate(
            [Alo.reshape(_N, _F), Blo.reshape(_N, _F), rep], axis=0)
        h = _edge_call(VAhi, VAlo, edge_attr, W_edge[l],
                       b_edge[l].reshape(1, _C), W_conv[l][:, 2 * _C:],
                       dst, src, lnw[l], lnb[l])
    return h.reshape(_N, _H, _C).transpose(0, 2, 1).reshape(_N, _F)


# speedup vs baseline: 5.5698x; 1.4046x over previous
import numpy as np
import jax
import jax.numpy as jnp
from jax import lax
from jax.experimental import pallas as pl
from jax.experimental.pallas import tpu as pltpu

_N = 384
_C = 64
_H = 60
_L = 4
_E = 2304
_ECH = 5
_F = _C * _H
_K = 2 * _N + _C

_EB = 576
_NCH = _E // _EB

_HIGHEST = jax.lax.Precision.HIGHEST


def _split16(v):
    hi = v.astype(jnp.bfloat16)
    lo = (v - hi.astype(jnp.float32)).astype(jnp.bfloat16)
    return hi, lo


def _gnn_kernel(xT_ref, eaT_ref, dst_ref, src_ref, wconv_ref, wedge_ref,
                bedgeT_ref, lnwT_ref, lnbT_ref, out_ref,
                hT_ref, vahi_ref, valo_ref):
    r_mod = lax.broadcasted_iota(jnp.int32, (_F, _C), 0) % _C
    r_col = lax.broadcasted_iota(jnp.int32, (_F, _C), 1)
    RT = (r_mod == r_col).astype(jnp.bfloat16)
    vahi_ref[:, 2 * _N:] = RT
    valo_ref[:, 2 * _N:] = RT

    iota_n = lax.broadcasted_iota(jnp.int32, (_N, _EB), 0)

    for l in range(_L):
        rd = xT_ref if l == 0 else hT_ref
        wr = out_ref if l == _L - 1 else hT_ref
        wconv = wconv_ref[l * _C:(l + 1) * _C, :]
        W1 = wconv[:, :_C]
        W2 = wconv[:, _C:2 * _C]
        W3 = wconv[:, 2 * _C:]
        MA = W1 - W2

        def _xform(i, carry):
            hsl = rd[pl.ds(i * _C, _C), :]
            asl = lax.dot(MA, hsl, precision=_HIGHEST,
                          preferred_element_type=jnp.float32)
            bsl = lax.dot(W2, hsl, precision=_HIGHEST,
                          preferred_element_type=jnp.float32)
            ahi, alo = _split16(asl)
            bhi, blo = _split16(bsl)
            vahi_ref[pl.ds(i * _C, _C), 0:_N] = ahi
            valo_ref[pl.ds(i * _C, _C), 0:_N] = alo
            vahi_ref[pl.ds(i * _C, _C), _N:2 * _N] = bhi
            valo_ref[pl.ds(i * _C, _C), _N:2 * _N] = blo
            return carry

        lax.fori_loop(0, _H, _xform, 0)

        vahi = vahi_ref[...]
        valo = valo_ref[...]
        for k in range(_NCH):
            dstr = dst_ref[0:1, k * _EB:(k + 1) * _EB]
            srcr = src_ref[0:1, k * _EB:(k + 1) * _EB]
            PdT = (iota_n == dstr).astype(jnp.bfloat16)
            PsT = (iota_n == srcr).astype(jnp.bfloat16)
            ea_c = eaT_ref[:, k * _EB:(k + 1) * _EB]
            tT = lax.dot(wedge_ref[l * _C:(l + 1) * _C, :], ea_c,
                         precision=_HIGHEST,
                         preferred_element_type=jnp.float32)
            tT = tT + bedgeT_ref[:, l:l + 1]
            cT = lax.dot(W3, tT, precision=_HIGHEST,
                         preferred_element_type=jnp.float32)
            chi, clo = _split16(cT)
            rhs_hi = jnp.concatenate([PdT, PsT, chi], axis=0)
            rhs_lo = jnp.concatenate([PdT, PsT, clo], axis=0)
            m = (lax.dot(vahi, rhs_hi, preferred_element_type=jnp.float32)
                 + lax.dot(valo, rhs_lo,
                           preferred_element_type=jnp.float32))
            mu = jnp.mean(m, axis=0, keepdims=True)
            d = m - mu
            var = jnp.mean(d * d, axis=0, keepdims=True)
            mn = d * lax.rsqrt(var + 1e-5)
            mn = mn * lnwT_ref[:, l:l + 1] + lnbT_ref[:, l:l + 1]
            mact = jnp.maximum(mn, 0.0)
            ahi, alo = _split16(mact)
            contrib = (lax.dot_general(ahi, PdT, (((1,), (1,)), ((), ())),
                                       preferred_element_type=jnp.float32)
                       + lax.dot_general(alo, PdT,
                                         (((1,), (1,)), ((), ())),
                                         preferred_element_type=jnp.float32))
            if k == 0:
                wr[...] = contrib
            else:
                wr[...] += contrib


_gnn_call = pl.pallas_call(
    _gnn_kernel,
    out_shape=jax.ShapeDtypeStruct((_F, _N), jnp.float32),
    scratch_shapes=[
        pltpu.VMEM((_F, _N), jnp.float32),
        pltpu.VMEM((_F, _K), jnp.bfloat16),
        pltpu.VMEM((_F, _K), jnp.bfloat16),
    ],
)


def kernel(x, edge_attr, edge_index, W_edge, b_edge, W_conv, ln_w, ln_b):
    xT = x.reshape(_N, _C, _H).transpose(2, 1, 0).reshape(_F, _N)
    eaT = edge_attr.T
    src = jnp.broadcast_to(edge_index[0].reshape(1, _E), (8, _E))
    dst = jnp.broadcast_to(edge_index[1].reshape(1, _E), (8, _E))
    wconv = W_conv.reshape(_L * _C, 3 * _C)
    wedge = W_edge.reshape(_L * _C, _ECH)
    bedgeT = b_edge.T
    lnwT = jnp.transpose(ln_w, (2, 1, 0)).reshape(_F, _L)
    lnbT = jnp.transpose(ln_b, (2, 1, 0)).reshape(_F, _L)
    outT = _gnn_call(xT, eaT, dst, src, wconv, wedge, bedgeT, lnwT, lnbT)
    return outT.reshape(_H, _C, _N).transpose(2, 1, 0).reshape(_N, _F)


# speedup vs baseline: 6.6574x; 1.1953x over previous
import numpy as np
import jax
import jax.numpy as jnp
from jax import lax
from jax.experimental import pallas as pl
from jax.experimental.pallas import tpu as pltpu

_N = 384
_C = 64
_H = 60
_L = 4
_E = 2304
_ECH = 5
_F = _C * _H
_K = 2 * _N + _C

_EB = 576
_NCH = _E // _EB

_HIGHEST = jax.lax.Precision.HIGHEST


def _split16(v):
    hi = v.astype(jnp.bfloat16)
    lo = (v - hi.astype(jnp.float32)).astype(jnp.bfloat16)
    return hi, lo


def _gnn_kernel(xT_ref, eaT_ref, dst_ref, src_ref, wconv_ref, wedge_ref,
                bedgeT_ref, lnwT_ref, lnbT_ref, out_ref,
                hT_ref, vahi_ref, valo_ref):
    r_mod = lax.broadcasted_iota(jnp.int32, (_F, _C), 0) % _C
    r_col = lax.broadcasted_iota(jnp.int32, (_F, _C), 1)
    RT = (r_mod == r_col).astype(jnp.bfloat16)
    vahi_ref[:, 2 * _N:] = RT
    valo_ref[:, 2 * _N:] = RT

    iota_n = lax.broadcasted_iota(jnp.int32, (_N, _EB), 0)

    for l in range(_L):
        rd = xT_ref if l == 0 else hT_ref
        wr = out_ref if l == _L - 1 else hT_ref
        wconv = wconv_ref[l * _C:(l + 1) * _C, :]
        W1 = wconv[:, :_C]
        W2 = wconv[:, _C:2 * _C]
        W3 = wconv[:, 2 * _C:]
        MA = W1 - W2

        def _xform(i, carry):
            hsl = rd[pl.ds(i * _C, _C), :]
            asl = lax.dot(MA, hsl, precision=_HIGHEST,
                          preferred_element_type=jnp.float32)
            bsl = lax.dot(W2, hsl, precision=_HIGHEST,
                          preferred_element_type=jnp.float32)
            ahi, alo = _split16(asl)
            bhi, blo = _split16(bsl)
            vahi_ref[pl.ds(i * _C, _C), 0:_N] = ahi
            valo_ref[pl.ds(i * _C, _C), 0:_N] = alo
            vahi_ref[pl.ds(i * _C, _C), _N:2 * _N] = bhi
            valo_ref[pl.ds(i * _C, _C), _N:2 * _N] = blo
            return carry

        lax.fori_loop(0, _H, _xform, 0)

        vahi = vahi_ref[...]
        valo = valo_ref[...]
        for k in range(_NCH):
            dstr = dst_ref[0:1, k * _EB:(k + 1) * _EB]
            srcr = src_ref[0:1, k * _EB:(k + 1) * _EB]
            PdT = (iota_n == dstr).astype(jnp.bfloat16)
            PsT = (iota_n == srcr).astype(jnp.bfloat16)
            ea_c = eaT_ref[:, k * _EB:(k + 1) * _EB]
            tT = lax.dot(wedge_ref[l * _C:(l + 1) * _C, :], ea_c,
                         precision=_HIGHEST,
                         preferred_element_type=jnp.float32)
            tT = tT + bedgeT_ref[:, l:l + 1]
            cT = lax.dot(W3, tT, precision=_HIGHEST,
                         preferred_element_type=jnp.float32)
            chi, clo = _split16(cT)
            rhs_hi = jnp.concatenate([PdT, PsT, chi], axis=0)
            m = (lax.dot(vahi, rhs_hi, preferred_element_type=jnp.float32)
                 + lax.dot(RT, clo,
                           preferred_element_type=jnp.float32))
            mu = jnp.mean(m, axis=0, keepdims=True)
            d = m - mu
            var = jnp.mean(d * d, axis=0, keepdims=True)
            mn = d * lax.rsqrt(var + 1e-5)
            mn = mn * lnwT_ref[:, l:l + 1] + lnbT_ref[:, l:l + 1]
            mact = jnp.maximum(mn, 0.0)
            ahi, alo = _split16(mact)
            contrib = (lax.dot_general(ahi, PdT, (((1,), (1,)), ((), ())),
                                       preferred_element_type=jnp.float32)
                       + lax.dot_general(alo, PdT,
                                         (((1,), (1,)), ((), ())),
                                         preferred_element_type=jnp.float32))
            if k == 0:
                wr[...] = contrib
            else:
                wr[...] += contrib


_gnn_call = pl.pallas_call(
    _gnn_kernel,
    out_shape=jax.ShapeDtypeStruct((_F, _N), jnp.float32),
    scratch_shapes=[
        pltpu.VMEM((_F, _N), jnp.float32),
        pltpu.VMEM((_F, _K), jnp.bfloat16),
        pltpu.VMEM((_F, _K), jnp.bfloat16),
    ],
)


def kernel(x, edge_attr, edge_index, W_edge, b_edge, W_conv, ln_w, ln_b):
    xT = x.reshape(_N, _C, _H).transpose(2, 1, 0).reshape(_F, _N)
    eaT = edge_attr.T
    src = jnp.broadcast_to(edge_index[0].reshape(1, _E), (8, _E))
    dst = jnp.broadcast_to(edge_index[1].reshape(1, _E), (8, _E))
    wconv = W_conv.reshape(_L * _C, 3 * _C)
    wedge = W_edge.reshape(_L * _C, _ECH)
    bedgeT = b_edge.T
    lnwT = jnp.transpose(ln_w, (2, 1, 0)).reshape(_F, _L)
    lnbT = jnp.transpose(ln_b, (2, 1, 0)).reshape(_F, _L)
    outT = _gnn_call(xT, eaT, dst, src, wconv, wedge, bedgeT, lnwT, lnbT)
    return outT.reshape(_H, _C, _N).transpose(2, 1, 0).reshape(_N, _F)
